# baseline (speedup 1.0000x reference)
import math

import jax
import jax.numpy as jnp
import numpy as np
from jax import lax
from jax.experimental import pallas as pl
from jax.experimental.pallas import tpu as pltpu
from jax.experimental.pallas import tpu_sc as plsc

DIM = 128
SEQ = 2048
BATCH = 4
SCALE = np.float32(math.sqrt(DIM))

NC = 2
NS = 16
NW = NC * NS
B = BATCH * SEQ
B_PER_W = B // NW
W_PER_SEQ = SEQ // B_PER_W
CHUNKS = (64, 128, 64)
OFFS = (0, 64, 192)
NCHUNK = len(CHUNKS)
LANES = 16

def _pe_table():
    position = np.arange(SEQ, dtype=np.float32)[:, None]
    div_term = np.exp(
        np.arange(0, DIM, 2, dtype=np.float32) * (-math.log(10000.0) / DIM))
    pe = np.zeros((SEQ, DIM), dtype=np.float32)
    pe[:, 0::2] = np.sin(position * div_term)
    pe[:, 1::2] = np.cos(position * div_term)
    return pe


_PE_REF = jax.new_ref(jnp.asarray(_pe_table()))


def _embed_body(idx_hbm, table_hbm, pe_hbm, out_hbm,
                idx_v, rows0, rows1, rows2, buf, sem_g, sem_pe, sem_o):
    rows = (rows0, rows1, rows2)
    wid = lax.axis_index("s") * NC + lax.axis_index("c")
    brow = wid // W_PER_SEQ
    pbase = lax.rem(wid, W_PER_SEQ) * B_PER_W
    base = wid * B_PER_W

    pes = [
        pltpu.async_copy(
            pe_hbm.at[pl.ds(pbase + OFFS[c], CHUNKS[c])],
            buf.at[pl.ds(OFFS[c], CHUNKS[c])], sem_pe.at[c])
        for c in range(NCHUNK)
    ]
    pltpu.sync_copy(idx_hbm.at[brow, pl.ds(pbase, B_PER_W)], idx_v)
    gathers = [
        pltpu.async_copy(table_hbm.at[idx_v.at[pl.ds(OFFS[c], CHUNKS[c])]],
                         rows[c], sem_g.at[c])
        for c in range(NCHUNK)
    ]

    outs = []
    for c in range(NCHUNK):
        pes[c].wait()
        gathers[c].wait()

        @plsc.parallel_loop(0, CHUNKS[c], unroll=4)
        def row(i):
            for j in range(DIM // LANES):
                sl = pl.ds(j * LANES, LANES)
                plsc.addupdate(buf.at[OFFS[c] + i, sl],
                               rows[c][i, sl] * SCALE)

        outs.append(pltpu.async_copy(
            buf.at[pl.ds(OFFS[c], CHUNKS[c])],
            out_hbm.at[pl.ds(base + OFFS[c], CHUNKS[c])], sem_o.at[c]))
    for co in outs:
        co.wait()


def kernel(x, table):
    call = pl.kernel(
        _embed_body,
        out_type=jax.ShapeDtypeStruct((B, DIM), jnp.float32),
        mesh=plsc.VectorSubcoreMesh(core_axis_name="c", subcore_axis_name="s"),
        scratch_types=[
            pltpu.VMEM((B_PER_W,), jnp.int32),
            pltpu.VMEM((CHUNKS[0], DIM), jnp.float32),
            pltpu.VMEM((CHUNKS[1], DIM), jnp.float32),
            pltpu.VMEM((CHUNKS[2], DIM), jnp.float32),
            pltpu.VMEM((B_PER_W, DIM), jnp.float32),
            pltpu.SemaphoreType.DMA((NCHUNK,)),
            pltpu.SemaphoreType.DMA((NCHUNK,)),
            pltpu.SemaphoreType.DMA((NCHUNK,)),
        ],
    )
    out = call(x, table, _PE_REF[...])
    return out.reshape(BATCH, SEQ, DIM)


# speedup vs baseline: 1.0431x; 1.0431x over previous
import math

import jax
import jax.numpy as jnp
import numpy as np
from jax import lax
from jax.experimental import pallas as pl
from jax.experimental.pallas import tpu as pltpu
from jax.experimental.pallas import tpu_sc as plsc

DIM = 128
SEQ = 2048
BATCH = 4
SCALE = np.float32(math.sqrt(DIM))

NC = 2
NS = 16
NW = NC * NS
B = BATCH * SEQ
B_PER_W = B // NW
W_PER_SEQ = SEQ // B_PER_W
CHUNKS = (64, 128, 64)
OFFS = (0, 64, 192)
NCHUNK = len(CHUNKS)
LANES = 16

def _pe_table():
    position = np.arange(SEQ, dtype=np.float32)[:, None]
    div_term = np.exp(
        np.arange(0, DIM, 2, dtype=np.float32) * (-math.log(10000.0) / DIM))
    pe = np.zeros((SEQ, DIM), dtype=np.float32)
    pe[:, 0::2] = np.sin(position * div_term)
    pe[:, 1::2] = np.cos(position * div_term)
    return pe


_PE_F16 = _pe_table().astype(np.float16)


def _pe_on_device():
    return lax.optimization_barrier(
        jnp.asarray(_PE_F16)).astype(jnp.float32)


def _embed_body(idx_hbm, table_hbm, pe_hbm, out_hbm,
                idx_v, rows0, rows1, rows2, buf, sem_g, sem_pe, sem_o):
    rows = (rows0, rows1, rows2)
    wid = lax.axis_index("s") * NC + lax.axis_index("c")
    brow = wid // W_PER_SEQ
    pbase = lax.rem(wid, W_PER_SEQ) * B_PER_W
    base = wid * B_PER_W

    pes = [
        pltpu.async_copy(
            pe_hbm.at[pl.ds(pbase + OFFS[c], CHUNKS[c])],
            buf.at[pl.ds(OFFS[c], CHUNKS[c])], sem_pe.at[c])
        for c in range(NCHUNK)
    ]
    with jax.named_scope("idx_copy"):
        pltpu.sync_copy(idx_hbm.at[brow, pl.ds(pbase, B_PER_W)], idx_v)
    gathers = [
        pltpu.async_copy(table_hbm.at[idx_v.at[pl.ds(OFFS[c], CHUNKS[c])]],
                         rows[c], sem_g.at[c])
        for c in range(NCHUNK)
    ]

    outs = []
    for c in range(NCHUNK):
        with jax.named_scope(f"wait{c}"):
            pes[c].wait()
            gathers[c].wait()

        with jax.named_scope(f"comp{c}"):
            @plsc.parallel_loop(0, CHUNKS[c], unroll=4)
            def row(i):
                for j in range(DIM // LANES):
                    sl = pl.ds(j * LANES, LANES)
                    plsc.addupdate(buf.at[OFFS[c] + i, sl],
                                   rows[c][i, sl] * SCALE)

        outs.append(pltpu.async_copy(
            buf.at[pl.ds(OFFS[c], CHUNKS[c])],
            out_hbm.at[pl.ds(base + OFFS[c], CHUNKS[c])], sem_o.at[c]))
    with jax.named_scope("drain"):
        for co in outs:
            co.wait()


def kernel(x, table):
    call = pl.kernel(
        _embed_body,
        out_type=jax.ShapeDtypeStruct((B, DIM), jnp.float32),
        mesh=plsc.VectorSubcoreMesh(core_axis_name="c", subcore_axis_name="s"),
        scratch_types=[
            pltpu.VMEM((B_PER_W,), jnp.int32),
            pltpu.VMEM((CHUNKS[0], DIM), jnp.float32),
            pltpu.VMEM((CHUNKS[1], DIM), jnp.float32),
            pltpu.VMEM((CHUNKS[2], DIM), jnp.float32),
            pltpu.VMEM((B_PER_W, DIM), jnp.float32),
            pltpu.SemaphoreType.DMA((NCHUNK,)),
            pltpu.SemaphoreType.DMA((NCHUNK,)),
            pltpu.SemaphoreType.DMA((NCHUNK,)),
        ],
    )
    out = call(x, table, _pe_on_device())
    return out.reshape(BATCH, SEQ, DIM)
